# baseline (speedup 1.0000x reference)
import functools
import math

import jax
import jax.numpy as jnp
from jax import lax
from jax.experimental import pallas as pl
from jax.experimental.pallas import tpu as pltpu
from jax.experimental.pallas import tpu_sc as plsc

_NC = 2
_NS = 16
_NW = _NC * _NS
_BLK = 64
_ROWB = 1000


def _leaky(x):
    return jnp.where(x >= 0, x, 0.01 * x)


def _proj(h, wq, bq, wk, bk, wv, bv, wsk, bsk, we2, qx_o, k_o, v_o, sk_o, d):
    q = jnp.dot(h, wq[...], preferred_element_type=jnp.float32) + bq[...]
    qx_o[:, :d] = q
    qx_o[:, d:] = lax.dot_general(q, we2[...], (((1,), (1,)), ((), ())),
                                  preferred_element_type=jnp.float32)
    k_o[...] = jnp.dot(h, wk[...], preferred_element_type=jnp.float32) + bk[...]
    v_o[...] = jnp.dot(h, wv[...], preferred_element_type=jnp.float32) + bv[...]
    sk_o[...] = jnp.dot(h, wsk[...], preferred_element_type=jnp.float32) + bsk[...]


def _tc_pre_body(x_ref, wq, bq, wk, bk, wv, bv, wsk, bsk, we,
                 qx_o, k_o, v_o, sk_o):
    d = x_ref.shape[1]
    _proj(x_ref[...], wq, bq, wk, bk, wv, bv, wsk, bsk, we,
          qx_o, k_o, v_o, sk_o, d)


def _tc_pre(x, p):
    n, d = x.shape
    hc = p['Wq'].shape[1]
    ed = p['We'].shape[0]
    grid = (n // _ROWB,)
    full = lambda *s: pl.BlockSpec(s, lambda i: (0,) * len(s))
    rb = pl.BlockSpec((_ROWB, d), lambda i: (i, 0))
    out_rb = pl.BlockSpec((_ROWB, hc), lambda i: (i, 0))
    return pl.pallas_call(
        _tc_pre_body,
        grid=grid,
        in_specs=[rb, full(d, hc), full(1, hc), full(d, hc), full(1, hc),
                  full(d, hc), full(1, hc), full(d, hc), full(1, hc),
                  full(ed, hc)],
        out_specs=[pl.BlockSpec((_ROWB, hc + ed), lambda i: (i, 0)),
                   out_rb, out_rb, out_rb],
        out_shape=[jax.ShapeDtypeStruct((n, hc + ed), jnp.float32)]
        + [jax.ShapeDtypeStruct((n, hc), jnp.float32)] * 3,
    )(x, p['Wq'], p['bq'].reshape(1, -1), p['Wk'], p['bk'].reshape(1, -1),
      p['Wv'], p['bv'].reshape(1, -1), p['Wskip'], p['bskip'].reshape(1, -1),
      p['We'])


def _combine(acc_ref, sk_ref, we_ref):
    d = sk_ref.shape[1]
    ed = we_ref.shape[0]
    a = acc_ref[0] + acc_ref[1]
    den = a[:, d + ed:d + ed + 1] + 1e-16
    h = (a[:, :d] + jnp.dot(a[:, d:d + ed], we_ref[...],
                            preferred_element_type=jnp.float32)) / den
    return _leaky(h + sk_ref[...])


def _tc_mid_body(acc_ref, sk_ref, we_ref,
                 wq, bq, wk, bk, wv, bv, wsk, bsk, we2,
                 h_o, qx_o, k_o, v_o, sk_o):
    h = _combine(acc_ref, sk_ref, we_ref)
    h_o[...] = h
    _proj(h, wq, bq, wk, bk, wv, bv, wsk, bsk, we2,
          qx_o, k_o, v_o, sk_o, sk_ref.shape[1])


def _tc_mid(acc, sk, we_prev, p, aw):
    n = sk.shape[0]
    d = sk.shape[1]
    hc = p['Wq'].shape[1]
    ed = we_prev.shape[0]
    grid = (n // _ROWB,)
    full = lambda *s: pl.BlockSpec(s, lambda i: (0,) * len(s))
    rb = pl.BlockSpec((_ROWB, d), lambda i: (i, 0))
    out_rb = pl.BlockSpec((_ROWB, hc), lambda i: (i, 0))
    return pl.pallas_call(
        _tc_mid_body,
        grid=grid,
        in_specs=[pl.BlockSpec((_NC, _ROWB, aw), lambda i: (0, i, 0)),
                  rb, full(ed, d),
                  full(d, hc), full(1, hc), full(d, hc), full(1, hc),
                  full(d, hc), full(1, hc), full(d, hc), full(1, hc),
                  full(ed, hc)],
        out_specs=[rb, pl.BlockSpec((_ROWB, hc + ed), lambda i: (i, 0)),
                   out_rb, out_rb, out_rb],
        out_shape=[jax.ShapeDtypeStruct((n, d), jnp.float32),
                   jax.ShapeDtypeStruct((n, hc + ed), jnp.float32)]
        + [jax.ShapeDtypeStruct((n, hc), jnp.float32)] * 3,
    )(acc, sk, we_prev, p['Wq'], p['bq'].reshape(1, -1),
      p['Wk'], p['bk'].reshape(1, -1), p['Wv'], p['bv'].reshape(1, -1),
      p['Wskip'], p['bskip'].reshape(1, -1), p['We'])


def _tc_mlp_body(h_ref, w1, b1, w2, b2, y_o):
    h = _leaky(jnp.dot(h_ref[...], w1[...],
                       preferred_element_type=jnp.float32) + b1[...])
    y_o[...] = jnp.dot(h, w2[...], preferred_element_type=jnp.float32) + b2[...]


def _tc_mlp(h, mlp):
    n, d = h.shape
    hid = mlp['W1'].shape[1]
    out = mlp['W2'].shape[1]
    grid = (n // _ROWB,)
    full = lambda *s: pl.BlockSpec(s, lambda i: (0,) * len(s))
    return pl.pallas_call(
        _tc_mlp_body,
        grid=grid,
        in_specs=[pl.BlockSpec((_ROWB, d), lambda i: (i, 0)),
                  full(d, hid), full(1, hid), full(hid, out), full(1, out)],
        out_specs=pl.BlockSpec((_ROWB, out), lambda i: (i, 0)),
        out_shape=jax.ShapeDtypeStruct((n, out), jnp.float32),
    )(h, mlp['W1'], mlp['b1'].reshape(1, -1),
      mlp['W2'], mlp['b2'].reshape(1, -1))


@functools.cache
def _make_edge_kernel(n, d, ed, e, ec_pad):
    nblk = ec_pad // _BLK
    half = n // 2
    qw = d + ed
    aw = d + ed + 8
    rpt = (half // _NS) // 8 * 8
    rem = half - rpt * _NS
    mesh = plsc.VectorSubcoreMesh(core_axis_name="c", subcore_axis_name="s",
                                  num_cores=_NC, num_subcores=_NS)
    inv = 1.0 / math.sqrt(d)

    @functools.partial(
        pl.kernel,
        out_type=jax.ShapeDtypeStruct((_NC, n, aw), jnp.float32),
        mesh=mesh,
        compiler_params=pltpu.CompilerParams(needs_layout_passes=False,
                                             use_tc_tiling_on_sc=False),
        scratch_types=[
            pltpu.VMEM((2, _BLK), jnp.int32),
            pltpu.VMEM((2, _BLK), jnp.int32),
            pltpu.VMEM((2, _BLK), jnp.int32),
            pltpu.VMEM((2, _BLK, qw), jnp.float32),
            pltpu.VMEM((2, _BLK, d), jnp.float32),
            pltpu.VMEM((2, _BLK, d), jnp.float32),
            pltpu.VMEM((2, _BLK, aw), jnp.float32),
            pltpu.VMEM((2, _BLK, ed), jnp.float32),
            pltpu.VMEM((nblk, _BLK), jnp.float32),
            pltpu.VMEM_SHARED((half, aw), jnp.float32),
            pltpu.SemaphoreType.DMA,
            pltpu.SemaphoreType.DMA,
            pltpu.SemaphoreType.DMA,
            pltpu.SemaphoreType.DMA,
            pltpu.SemaphoreType.DMA,
        ],
    )
    def edge_kernel(qx_hbm, k_hbm, v_hbm, ea_hbm, src_hbm, dst_hbm, zv_hbm,
                    acc_out,
                    src2, dst2, idxp, qxr, kr, vr, vw, ear, sbuf,
                    acc_sp, smq, smk, smv, smea, smsc):
        cid = lax.axis_index("c")
        sid = lax.axis_index("s")
        wid = cid * _NS + sid
        ebase = wid * ec_pad
        lane = lax.iota(jnp.int32, 16)
        zf = jnp.zeros((16,), jnp.float32)
        ngrp = _BLK // 16

        def zrow(i, c):
            p2v = jnp.full((16,), lax.div(i, ngrp), jnp.int32)
            rowi = lax.rem(i, ngrp) * 16 + lane
            for t in range(d + ed + 1, aw):
                plsc.store_scatter(
                    vw, [p2v, rowi, jnp.full((16,), t, jnp.int32)], zf)
            return c
        lax.fori_loop(0, 2 * ngrp, zrow, 0)

        def issue_gathers(jj, p):
            slot = lax.rem(jj, 2)
            pltpu.async_copy(v_hbm.at[src2.at[slot]], vr.at[slot], smv)
            if p == 0:
                pltpu.async_copy(qx_hbm.at[dst2.at[slot]], qxr.at[slot], smq)
                pltpu.async_copy(k_hbm.at[src2.at[slot]], kr.at[slot], smk)
            pltpu.async_copy(ea_hbm.at[pl.ds(ebase + jj * _BLK, _BLK)],
                             ear.at[slot], smea)

        def drain_gathers(p):
            pltpu.make_async_copy(v_hbm.at[pl.ds(0, _BLK)],
                                  vr.at[0], smv).wait()
            if p == 0:
                pltpu.make_async_copy(qx_hbm.at[pl.ds(0, _BLK)],
                                      qxr.at[0], smq).wait()
                pltpu.make_async_copy(k_hbm.at[pl.ds(0, _BLK)],
                                      kr.at[0], smk).wait()
            pltpu.make_async_copy(ea_hbm.at[pl.ds(0, _BLK)],
                                  ear.at[0], smea).wait()

        def drain_scatter():
            pltpu.make_async_copy(zv_hbm.at[pl.ds(0, _BLK)],
                                  vw.at[0], smsc).wait()

        for p in range(2):
            lo = p * half
            pltpu.sync_copy(zv_hbm.at[pl.ds(sid * rpt, rpt)],
                            acc_sp.at[pl.ds(sid * rpt, rpt)])
            if rem:
                @pl.when(sid == _NS - 1)
                def _zero_tail():
                    pltpu.sync_copy(zv_hbm.at[pl.ds(rpt * _NS, rem)],
                                    acc_sp.at[pl.ds(rpt * _NS, rem)])
            plsc.subcore_barrier()
            pltpu.sync_copy(src_hbm.at[wid, pl.ds(0, 2)], src2)
            pltpu.sync_copy(dst_hbm.at[wid, pl.ds(0, 2)], dst2)
            issue_gathers(0, p)

            def block(j, carry):
                par = lax.rem(j, 2)
                fpar = jnp.full((16,), par, jnp.int32)
                drain_gathers(p)

                @pl.when(j + 1 < nblk)
                def _issue_next():
                    issue_gathers(j + 1, p)

                @pl.when(j >= 2)
                def _drain_sc():
                    drain_scatter()

                gid0 = ebase + j * _BLK

                def grp(g, carry2):
                    rowi = g * 16 + lane
                    dstg = dst2[par, pl.ds(g * 16, 16)]
                    if p == 0:
                        @plsc.parallel_loop(0, d, step=16,
                                            carry=(zf, zf, zf, zf))
                        def featl(c0, accs):
                            aa = list(accs)
                            for t in range(16):
                                colv = jnp.full((16,), t, jnp.int32) + c0
                                pr = (plsc.load_gather(qxr, [fpar, rowi, colv])
                                      * plsc.load_gather(kr, [fpar, rowi, colv]))
                                aa[t % 4] = aa[t % 4] + pr
                            return tuple(aa)
                        b0, b1, b2, b3 = featl
                        a0, a1 = b0 + b2, b1 + b3

                        @plsc.parallel_loop(0, ed, step=8, carry=(a0, a1))
                        def featle(c0, accs):
                            a0, a1 = accs
                            for t in range(8):
                                colv = jnp.full((16,), t, jnp.int32) + c0
                                pr = (plsc.load_gather(
                                          qxr, [fpar, rowi, colv + d])
                                      * plsc.load_gather(ear, [fpar, rowi, colv]))
                                if t % 2 == 0:
                                    a0 = a0 + pr
                                else:
                                    a1 = a1 + pr
                            return (a0, a1)
                        a0, a1 = featle
                        ids = gid0 + rowi
                        sv = jnp.where(ids < e, jnp.exp((a0 + a1) * inv), 0.0)
                        sbuf[j, pl.ds(g * 16, 16)] = sv
                    else:
                        sv = sbuf[j, pl.ds(g * 16, 16)]
                    inb = (dstg >= lo) & (dstg < lo + half)
                    svp = jnp.where(inb, sv, 0.0)
                    idxp[par, pl.ds(g * 16, 16)] = jnp.where(inb, dstg - lo, 0)

                    @plsc.parallel_loop(0, d, step=16, carry=jnp.int32(0))
                    def _vcol(c0, c3):
                        for t in range(16):
                            colv = jnp.full((16,), t, jnp.int32) + c0
                            vv = plsc.load_gather(vr, [fpar, rowi, colv]) * svp
                            plsc.store_scatter(vw, [fpar, rowi, colv], vv)
                        return c3

                    @plsc.parallel_loop(0, ed, step=8, carry=jnp.int32(0))
                    def _ecol(c0, c3):
                        for t in range(8):
                            colv = jnp.full((16,), t, jnp.int32) + c0
                            ev = plsc.load_gather(ear, [fpar, rowi, colv]) * svp
                            plsc.store_scatter(vw, [fpar, rowi, colv + d], ev)
                        return c3
                    plsc.store_scatter(
                        vw, [fpar, rowi, jnp.full((16,), d + ed, jnp.int32)],
                        svp)
                    return carry2
                lax.fori_loop(0, ngrp, grp, 0)

                pltpu.async_copy(vw.at[par], acc_sp.at[idxp.at[par]], smsc,
                                 add=True)
                @pl.when(j + 2 < nblk)
                def _load_next():
                    pltpu.sync_copy(src_hbm.at[wid, j + 2], src2.at[par])
                    pltpu.sync_copy(dst_hbm.at[wid, j + 2], dst2.at[par])
                return carry
            lax.fori_loop(0, nblk, block, 0)
            drain_scatter()
            drain_scatter()
            plsc.subcore_barrier()
            pltpu.sync_copy(acc_sp.at[pl.ds(sid * rpt, rpt)],
                            acc_out.at[cid, pl.ds(lo + sid * rpt, rpt)])
            if rem:
                @pl.when(sid == _NS - 1)
                def _spill_tail():
                    pltpu.sync_copy(
                        acc_sp.at[pl.ds(rpt * _NS, rem)],
                        acc_out.at[cid, pl.ds(lo + rpt * _NS, rem)])

    return edge_kernel


def kernel(x, pe, edge_index, edge_attr, batch, params):
    n, d = x.shape
    e = edge_index.shape[1]
    ed = edge_attr.shape[1]
    aw = d + ed + 8
    layers = params['layers']
    ec_pad = -(-e // (_NW * 2 * _BLK)) * 2 * _BLK
    pad = ec_pad * _NW - e

    src_r = jnp.pad(edge_index[0], (0, pad)).reshape(_NW, ec_pad // _BLK, _BLK)
    dst_r = jnp.pad(edge_index[1], (0, pad)).reshape(_NW, ec_pad // _BLK, _BLK)
    ea_pad = jnp.pad(edge_attr, ((0, pad), (0, 0)))
    zv = jnp.zeros((n, aw), jnp.float32)

    edge_fn = _make_edge_kernel(n, d, ed, e, ec_pad)

    p1, p2 = layers[0], layers[1]
    qx, k, v, sk = _tc_pre(x, p1)
    ws = {'We_comb': jnp.stack([p1['We'], p2['We']])}
    for name in ('Wq', 'bq', 'Wk', 'bk', 'Wv', 'bv', 'Wskip', 'bskip', 'We'):
        ws[name] = jnp.stack([p2[name], p2[name]])

    def step(carry, w):
        qx, k, v, sk, _ = carry
        acc = edge_fn(qx, k, v, ea_pad, src_r, dst_r, zv)
        h, qx2, k2, v2, sk2 = _tc_mid(acc, sk, w['We_comb'], w, aw)
        return (qx2, k2, v2, sk2, h), None

    carry, _ = lax.scan(step, (qx, k, v, sk, x), ws)
    return _tc_mlp(carry[4], params['mlp'])


# speedup vs baseline: 2.2657x; 2.2657x over previous
import functools
import math

import jax
import jax.numpy as jnp
from jax import lax
from jax.experimental import pallas as pl
from jax.experimental.pallas import tpu as pltpu
from jax.experimental.pallas import tpu_sc as plsc

_NC = 2
_NS = 16
_NW = _NC * _NS
_BLK = 64
_ROWB = 1000
_BCAST_DNUMS = lax.GatherDimensionNumbers(
    offset_dims=(), collapsed_slice_dims=(0,), start_index_map=(0,))


def _leaky(x):
    return jnp.where(x >= 0, x, 0.01 * x)


def _proj(h, wq, bq, wk, bk, wv, bv, wsk, bsk, we2, qx_o, k_o, v_o, sk_o, d):
    q = jnp.dot(h, wq[...], preferred_element_type=jnp.float32) + bq[...]
    qx_o[:, :d] = q
    qx_o[:, d:] = lax.dot_general(q, we2[...], (((1,), (1,)), ((), ())),
                                  preferred_element_type=jnp.float32)
    k_o[...] = jnp.dot(h, wk[...], preferred_element_type=jnp.float32) + bk[...]
    v_o[...] = jnp.dot(h, wv[...], preferred_element_type=jnp.float32) + bv[...]
    sk_o[...] = jnp.dot(h, wsk[...], preferred_element_type=jnp.float32) + bsk[...]


def _tc_pre_body(x_ref, wq, bq, wk, bk, wv, bv, wsk, bsk, we,
                 qx_o, k_o, v_o, sk_o):
    d = x_ref.shape[1]
    _proj(x_ref[...], wq, bq, wk, bk, wv, bv, wsk, bsk, we,
          qx_o, k_o, v_o, sk_o, d)


def _tc_pre(x, p):
    n, d = x.shape
    hc = p['Wq'].shape[1]
    ed = p['We'].shape[0]
    grid = (n // _ROWB,)
    full = lambda *s: pl.BlockSpec(s, lambda i: (0,) * len(s))
    rb = pl.BlockSpec((_ROWB, d), lambda i: (i, 0))
    out_rb = pl.BlockSpec((_ROWB, hc), lambda i: (i, 0))
    return pl.pallas_call(
        _tc_pre_body,
        grid=grid,
        in_specs=[rb, full(d, hc), full(1, hc), full(d, hc), full(1, hc),
                  full(d, hc), full(1, hc), full(d, hc), full(1, hc),
                  full(ed, hc)],
        out_specs=[pl.BlockSpec((_ROWB, hc + ed), lambda i: (i, 0)),
                   out_rb, out_rb, out_rb],
        out_shape=[jax.ShapeDtypeStruct((n, hc + ed), jnp.float32)]
        + [jax.ShapeDtypeStruct((n, hc), jnp.float32)] * 3,
    )(x, p['Wq'], p['bq'].reshape(1, -1), p['Wk'], p['bk'].reshape(1, -1),
      p['Wv'], p['bv'].reshape(1, -1), p['Wskip'], p['bskip'].reshape(1, -1),
      p['We'])


def _combine(acc_ref, sk_ref, we_ref):
    d = sk_ref.shape[1]
    ed = we_ref.shape[0]
    a = acc_ref[0] + acc_ref[1]
    den = a[:, d + ed:d + ed + 1] + 1e-16
    h = (a[:, :d] + jnp.dot(a[:, d:d + ed], we_ref[...],
                            preferred_element_type=jnp.float32)) / den
    return _leaky(h + sk_ref[...])


def _tc_mid_body(acc_ref, sk_ref, we_ref,
                 wq, bq, wk, bk, wv, bv, wsk, bsk, we2,
                 h_o, qx_o, k_o, v_o, sk_o):
    h = _combine(acc_ref, sk_ref, we_ref)
    h_o[...] = h
    _proj(h, wq, bq, wk, bk, wv, bv, wsk, bsk, we2,
          qx_o, k_o, v_o, sk_o, sk_ref.shape[1])


def _tc_mid(acc, sk, we_prev, p, aw):
    n = sk.shape[0]
    d = sk.shape[1]
    hc = p['Wq'].shape[1]
    ed = we_prev.shape[0]
    grid = (n // _ROWB,)
    full = lambda *s: pl.BlockSpec(s, lambda i: (0,) * len(s))
    rb = pl.BlockSpec((_ROWB, d), lambda i: (i, 0))
    out_rb = pl.BlockSpec((_ROWB, hc), lambda i: (i, 0))
    return pl.pallas_call(
        _tc_mid_body,
        grid=grid,
        in_specs=[pl.BlockSpec((_NC, _ROWB, aw), lambda i: (0, i, 0)),
                  rb, full(ed, d),
                  full(d, hc), full(1, hc), full(d, hc), full(1, hc),
                  full(d, hc), full(1, hc), full(d, hc), full(1, hc),
                  full(ed, hc)],
        out_specs=[rb, pl.BlockSpec((_ROWB, hc + ed), lambda i: (i, 0)),
                   out_rb, out_rb, out_rb],
        out_shape=[jax.ShapeDtypeStruct((n, d), jnp.float32),
                   jax.ShapeDtypeStruct((n, hc + ed), jnp.float32)]
        + [jax.ShapeDtypeStruct((n, hc), jnp.float32)] * 3,
    )(acc, sk, we_prev, p['Wq'], p['bq'].reshape(1, -1),
      p['Wk'], p['bk'].reshape(1, -1), p['Wv'], p['bv'].reshape(1, -1),
      p['Wskip'], p['bskip'].reshape(1, -1), p['We'])


def _tc_mlp_body(h_ref, w1, b1, w2, b2, y_o):
    h = _leaky(jnp.dot(h_ref[...], w1[...],
                       preferred_element_type=jnp.float32) + b1[...])
    y_o[...] = jnp.dot(h, w2[...], preferred_element_type=jnp.float32) + b2[...]


def _tc_mlp(h, mlp):
    n, d = h.shape
    hid = mlp['W1'].shape[1]
    out = mlp['W2'].shape[1]
    grid = (n // _ROWB,)
    full = lambda *s: pl.BlockSpec(s, lambda i: (0,) * len(s))
    return pl.pallas_call(
        _tc_mlp_body,
        grid=grid,
        in_specs=[pl.BlockSpec((_ROWB, d), lambda i: (i, 0)),
                  full(d, hid), full(1, hid), full(hid, out), full(1, out)],
        out_specs=pl.BlockSpec((_ROWB, out), lambda i: (i, 0)),
        out_shape=jax.ShapeDtypeStruct((n, out), jnp.float32),
    )(h, mlp['W1'], mlp['b1'].reshape(1, -1),
      mlp['W2'], mlp['b2'].reshape(1, -1))


@functools.cache
def _make_edge_kernel(n, d, ed, e, ec_pad):
    nblk = ec_pad // _BLK
    half = n // 2
    qw = d + ed
    aw = d + ed + 8
    rpt = (half // _NS) // 8 * 8
    rem = half - rpt * _NS
    mesh = plsc.VectorSubcoreMesh(core_axis_name="c", subcore_axis_name="s",
                                  num_cores=_NC, num_subcores=_NS)
    inv = 1.0 / math.sqrt(d)

    @functools.partial(
        pl.kernel,
        out_type=jax.ShapeDtypeStruct((_NC, n, aw), jnp.float32),
        mesh=mesh,
        compiler_params=pltpu.CompilerParams(needs_layout_passes=False,
                                             use_tc_tiling_on_sc=False),
        scratch_types=[
            pltpu.VMEM((6, _BLK), jnp.int32),
            pltpu.VMEM((2, _BLK, qw), jnp.float32),
            pltpu.VMEM((2, _BLK, d), jnp.float32),
            pltpu.VMEM((2, _BLK, d), jnp.float32),
            pltpu.VMEM((2, _BLK, aw), jnp.float32),
            pltpu.VMEM((nblk, _BLK), jnp.float32),
            pltpu.VMEM((16, 17), jnp.float32),
            pltpu.VMEM_SHARED((half, aw), jnp.float32),
            pltpu.SemaphoreType.DMA,
            pltpu.SemaphoreType.DMA,
            pltpu.SemaphoreType.DMA,
            pltpu.SemaphoreType.DMA,
            pltpu.SemaphoreType.DMA,
        ],
    )
    def edge_kernel(qx_hbm, k_hbm, v_hbm, ea_hbm, src_hbm, dst_hbm, zv_hbm,
                    acc_out,
                    idx6, qxr, kr, vr, vw, sbuf, pbuf,
                    acc_sp, smq, smk, smv, smea, smsc):
        cid = lax.axis_index("c")
        sid = lax.axis_index("s")
        wid = cid * _NS + sid
        ebase = wid * ec_pad
        lane = lax.iota(jnp.int32, 16)
        zf = jnp.zeros((16,), jnp.float32)
        ngrp = _BLK // 16

        def zrow(i, c):
            p2v = jnp.full((16,), lax.div(i, ngrp), jnp.int32)
            rowi = lax.rem(i, ngrp) * 16 + lane
            for t in range(d + ed + 1, aw):
                plsc.store_scatter(
                    vw, [p2v, rowi, jnp.full((16,), t, jnp.int32)], zf)
            return c
        lax.fori_loop(0, 2 * ngrp, zrow, 0)

        def issue_gathers(jj, p):
            slot = lax.rem(jj, 2)
            pltpu.async_copy(v_hbm.at[idx6.at[slot]], vr.at[slot], smv)
            if p == 0:
                pltpu.async_copy(qx_hbm.at[idx6.at[2 + slot]], qxr.at[slot], smq)
                pltpu.async_copy(k_hbm.at[idx6.at[slot]], kr.at[slot], smk)
            pltpu.async_copy(ea_hbm.at[pl.ds(ebase + jj * _BLK, _BLK)],
                             vw.at[slot, :, pl.ds(d, ed)], smea)

        def drain_gathers(p):
            pltpu.make_async_copy(v_hbm.at[pl.ds(0, _BLK)],
                                  vr.at[0], smv).wait()
            if p == 0:
                pltpu.make_async_copy(qx_hbm.at[pl.ds(0, _BLK)],
                                      qxr.at[0], smq).wait()
                pltpu.make_async_copy(k_hbm.at[pl.ds(0, _BLK)],
                                      kr.at[0], smk).wait()
            pltpu.make_async_copy(ea_hbm.at[pl.ds(0, _BLK)],
                                  vw.at[0, :, pl.ds(d, ed)], smea).wait()

        def drain_scatter():
            pltpu.make_async_copy(zv_hbm.at[pl.ds(0, _BLK)],
                                  vw.at[0], smsc).wait()

        for p in range(2):
            lo = p * half
            pltpu.sync_copy(zv_hbm.at[pl.ds(sid * rpt, rpt)],
                            acc_sp.at[pl.ds(sid * rpt, rpt)])
            if rem:
                @pl.when(sid == _NS - 1)
                def _zero_tail():
                    pltpu.sync_copy(zv_hbm.at[pl.ds(rpt * _NS, rem)],
                                    acc_sp.at[pl.ds(rpt * _NS, rem)])
            plsc.subcore_barrier()
            pltpu.sync_copy(src_hbm.at[wid, pl.ds(0, 2)], idx6.at[pl.ds(0, 2)])
            pltpu.sync_copy(dst_hbm.at[wid, pl.ds(0, 2)], idx6.at[pl.ds(2, 2)])
            issue_gathers(0, p)

            def block(j, carry):
                par = lax.rem(j, 2)
                fpar = jnp.full((16,), par, jnp.int32)
                drain_gathers(p)

                @pl.when(j >= 1)
                def _drain_sc():
                    drain_scatter()

                @pl.when(j + 1 < nblk)
                def _issue_next():
                    issue_gathers(j + 1, p)

                gid0 = ebase + j * _BLK

                def grp(g, carry2):
                    g16 = g * 16
                    rowi = g16 + lane
                    dstg = idx6[2 + par, pl.ds(g16, 16)]
                    if p == 0:
                        @plsc.parallel_loop(0, 16, step=2, carry=jnp.int32(0))
                        def dotl(i0, c3):
                            for t in range(2):
                                i = i0 + t
                                r = g16 + i
                                a0 = (qxr[par, r, pl.ds(0, 16)]
                                      * kr[par, r, pl.ds(0, 16)])
                                a1 = (qxr[par, r, pl.ds(16, 16)]
                                      * kr[par, r, pl.ds(16, 16)])
                                for c in range(2, d // 16):
                                    pr = (qxr[par, r, pl.ds(c * 16, 16)]
                                          * kr[par, r, pl.ds(c * 16, 16)])
                                    if c % 2 == 0:
                                        a0 = a0 + pr
                                    else:
                                        a1 = a1 + pr
                                a0 = a0 + (qxr[par, r, pl.ds(d, 16)]
                                           * vw[par, r, pl.ds(d, 16)])
                                pbuf[i, pl.ds(0, 16)] = a0 + a1
                            return c3
                        sv0 = zf
                        sv1 = zf
                        for l in range(0, 16, 2):
                            sv0 = sv0 + plsc.load_gather(
                                pbuf, [lane, jnp.full((16,), l, jnp.int32)])
                            sv1 = sv1 + plsc.load_gather(
                                pbuf, [lane, jnp.full((16,), l + 1, jnp.int32)])
                        ids = gid0 + rowi
                        sv = jnp.where(ids < e,
                                       jnp.exp((sv0 + sv1) * inv), 0.0)
                        sbuf[j, pl.ds(g16, 16)] = sv
                    else:
                        sv = sbuf[j, pl.ds(g16, 16)]
                    inb = (dstg >= lo) & (dstg < lo + half)
                    svp = jnp.where(inb, sv, 0.0)
                    idx6[4 + par, pl.ds(g16, 16)] = jnp.where(inb, dstg - lo, 0)

                    @plsc.parallel_loop(0, 16, step=2, carry=jnp.int32(0))
                    def scl(i0, c3):
                        for t in range(2):
                            i = i0 + t
                            r = g16 + i
                            bc = lax.gather(
                                svp, jnp.full((16, 1), i, jnp.int32),
                                _BCAST_DNUMS, (1,),
                                mode=lax.GatherScatterMode.PROMISE_IN_BOUNDS)
                            for c in range(d // 16):
                                vw[par, r, pl.ds(c * 16, 16)] = (
                                    vr[par, r, pl.ds(c * 16, 16)] * bc)
                            vw[par, r, pl.ds(d, 16)] = (
                                vw[par, r, pl.ds(d, 16)] * bc)
                        return c3
                    plsc.store_scatter(
                        vw, [fpar, rowi, jnp.full((16,), d + ed, jnp.int32)],
                        svp)
                    return carry2
                lax.fori_loop(0, ngrp, grp, 0)

                pltpu.async_copy(vw.at[par], acc_sp.at[idx6.at[4 + par]], smsc,
                                 add=True)
                @pl.when(j + 2 < nblk)
                def _load_next():
                    pltpu.sync_copy(src_hbm.at[wid, j + 2], idx6.at[par])
                    pltpu.sync_copy(dst_hbm.at[wid, j + 2], idx6.at[2 + par])
                return carry
            lax.fori_loop(0, nblk, block, 0)
            drain_scatter()
            plsc.subcore_barrier()
            pltpu.sync_copy(acc_sp.at[pl.ds(sid * rpt, rpt)],
                            acc_out.at[cid, pl.ds(lo + sid * rpt, rpt)])
            if rem:
                @pl.when(sid == _NS - 1)
                def _spill_tail():
                    pltpu.sync_copy(
                        acc_sp.at[pl.ds(rpt * _NS, rem)],
                        acc_out.at[cid, pl.ds(lo + rpt * _NS, rem)])

    return edge_kernel


def kernel(x, pe, edge_index, edge_attr, batch, params):
    n, d = x.shape
    e = edge_index.shape[1]
    ed = edge_attr.shape[1]
    aw = d + ed + 8
    layers = params['layers']
    ec_pad = -(-e // (_NW * 2 * _BLK)) * 2 * _BLK
    pad = ec_pad * _NW - e

    src_r = jnp.pad(edge_index[0], (0, pad)).reshape(_NW, ec_pad // _BLK, _BLK)
    dst_r = jnp.pad(edge_index[1], (0, pad)).reshape(_NW, ec_pad // _BLK, _BLK)
    ea_pad = jnp.pad(edge_attr, ((0, pad), (0, 0)))
    zv = jnp.zeros((n, aw), jnp.float32)

    edge_fn = _make_edge_kernel(n, d, ed, e, ec_pad)

    p1, p2 = layers[0], layers[1]
    qx, k, v, sk = _tc_pre(x, p1)
    ws = {'We_comb': jnp.stack([p1['We'], p2['We']])}
    for name in ('Wq', 'bq', 'Wk', 'bk', 'Wv', 'bv', 'Wskip', 'bskip', 'We'):
        ws[name] = jnp.stack([p2[name], p2[name]])

    def step(carry, w):
        qx, k, v, sk, _ = carry
        acc = edge_fn(qx, k, v, ea_pad, src_r, dst_r, zv)
        h, qx2, k2, v2, sk2 = _tc_mid(acc, sk, w['We_comb'], w, aw)
        return (qx2, k2, v2, sk2, h), None

    carry, _ = lax.scan(step, (qx, k, v, sk, x), ws)
    return _tc_mlp(carry[4], params['mlp'])


# speedup vs baseline: 2.4289x; 1.0720x over previous
import functools
import math

import jax
import jax.numpy as jnp
from jax import lax
from jax.experimental import pallas as pl
from jax.experimental.pallas import tpu as pltpu
from jax.experimental.pallas import tpu_sc as plsc

_NC = 2
_NS = 16
_NW = _NC * _NS
_BLK = 64
_ROWB = 1000
_BCAST_DNUMS = lax.GatherDimensionNumbers(
    offset_dims=(), collapsed_slice_dims=(0,), start_index_map=(0,))


def _leaky(x):
    return jnp.where(x >= 0, x, 0.01 * x)


def _proj(h, wq, bq, wk, bk, wv, bv, wsk, bsk, we2, qx_o, k_o, v_o, sk_o, d):
    q = jnp.dot(h, wq[...], preferred_element_type=jnp.float32) + bq[...]
    qx_o[:, :d] = q
    qx_o[:, d:] = lax.dot_general(q, we2[...], (((1,), (1,)), ((), ())),
                                  preferred_element_type=jnp.float32)
    k_o[...] = jnp.dot(h, wk[...], preferred_element_type=jnp.float32) + bk[...]
    v_o[...] = jnp.dot(h, wv[...], preferred_element_type=jnp.float32) + bv[...]
    sk_o[...] = jnp.dot(h, wsk[...], preferred_element_type=jnp.float32) + bsk[...]


def _tc_pre_body(x_ref, wq, bq, wk, bk, wv, bv, wsk, bsk, we,
                 qx_o, k_o, v_o, sk_o):
    d = x_ref.shape[1]
    _proj(x_ref[...], wq, bq, wk, bk, wv, bv, wsk, bsk, we,
          qx_o, k_o, v_o, sk_o, d)


def _tc_pre(x, p):
    n, d = x.shape
    hc = p['Wq'].shape[1]
    ed = p['We'].shape[0]
    grid = (n // _ROWB,)
    full = lambda *s: pl.BlockSpec(s, lambda i: (0,) * len(s))
    rb = pl.BlockSpec((_ROWB, d), lambda i: (i, 0))
    out_rb = pl.BlockSpec((_ROWB, hc), lambda i: (i, 0))
    return pl.pallas_call(
        _tc_pre_body,
        grid=grid,
        in_specs=[rb, full(d, hc), full(1, hc), full(d, hc), full(1, hc),
                  full(d, hc), full(1, hc), full(d, hc), full(1, hc),
                  full(ed, hc)],
        out_specs=[pl.BlockSpec((_ROWB, hc + ed), lambda i: (i, 0)),
                   out_rb, out_rb, out_rb],
        out_shape=[jax.ShapeDtypeStruct((n, hc + ed), jnp.float32)]
        + [jax.ShapeDtypeStruct((n, hc), jnp.float32)] * 3,
    )(x, p['Wq'], p['bq'].reshape(1, -1), p['Wk'], p['bk'].reshape(1, -1),
      p['Wv'], p['bv'].reshape(1, -1), p['Wskip'], p['bskip'].reshape(1, -1),
      p['We'])


def _combine(acc_ref, sk_ref, we_ref):
    d = sk_ref.shape[1]
    ed = we_ref.shape[0]
    a = acc_ref[0] + acc_ref[1]
    den = a[:, d + ed:d + ed + 1] + 1e-16
    h = (a[:, :d] + jnp.dot(a[:, d:d + ed], we_ref[...],
                            preferred_element_type=jnp.float32)) / den
    return _leaky(h + sk_ref[...])


def _tc_mid_body(acc_ref, sk_ref, we_ref,
                 wq, bq, wk, bk, wv, bv, wsk, bsk, we2,
                 h_o, qx_o, k_o, v_o, sk_o):
    h = _combine(acc_ref, sk_ref, we_ref)
    h_o[...] = h
    _proj(h, wq, bq, wk, bk, wv, bv, wsk, bsk, we2,
          qx_o, k_o, v_o, sk_o, sk_ref.shape[1])


def _tc_mid(acc, sk, we_prev, p, aw):
    n = sk.shape[0]
    d = sk.shape[1]
    hc = p['Wq'].shape[1]
    ed = we_prev.shape[0]
    grid = (n // _ROWB,)
    full = lambda *s: pl.BlockSpec(s, lambda i: (0,) * len(s))
    rb = pl.BlockSpec((_ROWB, d), lambda i: (i, 0))
    out_rb = pl.BlockSpec((_ROWB, hc), lambda i: (i, 0))
    return pl.pallas_call(
        _tc_mid_body,
        grid=grid,
        in_specs=[pl.BlockSpec((_NC, _ROWB, aw), lambda i: (0, i, 0)),
                  rb, full(ed, d),
                  full(d, hc), full(1, hc), full(d, hc), full(1, hc),
                  full(d, hc), full(1, hc), full(d, hc), full(1, hc),
                  full(ed, hc)],
        out_specs=[rb, pl.BlockSpec((_ROWB, hc + ed), lambda i: (i, 0)),
                   out_rb, out_rb, out_rb],
        out_shape=[jax.ShapeDtypeStruct((n, d), jnp.float32),
                   jax.ShapeDtypeStruct((n, hc + ed), jnp.float32)]
        + [jax.ShapeDtypeStruct((n, hc), jnp.float32)] * 3,
    )(acc, sk, we_prev, p['Wq'], p['bq'].reshape(1, -1),
      p['Wk'], p['bk'].reshape(1, -1), p['Wv'], p['bv'].reshape(1, -1),
      p['Wskip'], p['bskip'].reshape(1, -1), p['We'])


def _tc_mlp_body(h_ref, w1, b1, w2, b2, y_o):
    h = _leaky(jnp.dot(h_ref[...], w1[...],
                       preferred_element_type=jnp.float32) + b1[...])
    y_o[...] = jnp.dot(h, w2[...], preferred_element_type=jnp.float32) + b2[...]


def _tc_mlp(h, mlp):
    n, d = h.shape
    hid = mlp['W1'].shape[1]
    out = mlp['W2'].shape[1]
    grid = (n // _ROWB,)
    full = lambda *s: pl.BlockSpec(s, lambda i: (0,) * len(s))
    return pl.pallas_call(
        _tc_mlp_body,
        grid=grid,
        in_specs=[pl.BlockSpec((_ROWB, d), lambda i: (i, 0)),
                  full(d, hid), full(1, hid), full(hid, out), full(1, out)],
        out_specs=pl.BlockSpec((_ROWB, out), lambda i: (i, 0)),
        out_shape=jax.ShapeDtypeStruct((n, out), jnp.float32),
    )(h, mlp['W1'], mlp['b1'].reshape(1, -1),
      mlp['W2'], mlp['b2'].reshape(1, -1))


@functools.cache
def _make_edge_kernel(n, d, ed, e, ec_pad):
    nblk = ec_pad // _BLK
    half = n // 2
    qw = d + ed
    aw = d + ed + 8
    rpt = (half // _NS) // 8 * 8
    rem = half - rpt * _NS
    mesh = plsc.VectorSubcoreMesh(core_axis_name="c", subcore_axis_name="s",
                                  num_cores=_NC, num_subcores=_NS)
    inv = 1.0 / math.sqrt(d)

    @functools.partial(
        pl.kernel,
        out_type=jax.ShapeDtypeStruct((_NC, n, aw), jnp.float32),
        mesh=mesh,
        compiler_params=pltpu.CompilerParams(needs_layout_passes=False,
                                             use_tc_tiling_on_sc=False),
        scratch_types=[
            pltpu.VMEM((7, _BLK), jnp.int32),
            pltpu.VMEM((2, _BLK, qw), jnp.float32),
            pltpu.VMEM((2, _BLK, d), jnp.float32),
            pltpu.VMEM((2, _BLK, d), jnp.float32),
            pltpu.VMEM((2, _BLK, aw), jnp.float32),
            pltpu.VMEM((nblk, _BLK), jnp.float32),
            pltpu.VMEM((16, 17), jnp.float32),
            pltpu.VMEM_SHARED((half, aw), jnp.float32),
            pltpu.SemaphoreType.DMA,
            pltpu.SemaphoreType.DMA,
            pltpu.SemaphoreType.DMA,
            pltpu.SemaphoreType.DMA,
            pltpu.SemaphoreType.DMA,
            pltpu.SemaphoreType.DMA,
            pltpu.SemaphoreType.DMA,
        ],
    )
    def edge_kernel(qx_hbm, k_hbm, v_hbm, ea_hbm, src_hbm, dst_hbm, zv_hbm,
                    acc_out,
                    idx6, qxr, kr, vr, vw, sbuf, pbuf,
                    acc_sp, smq, smk, smv, smea, smsc, smsi, smdi):
        cid = lax.axis_index("c")
        sid = lax.axis_index("s")
        wid = cid * _NS + sid
        ebase = wid * ec_pad
        lane = lax.iota(jnp.int32, 16)
        zf = jnp.zeros((16,), jnp.float32)
        ngrp = _BLK // 16

        def zrow(i, c):
            p2v = jnp.full((16,), lax.div(i, ngrp), jnp.int32)
            rowi = lax.rem(i, ngrp) * 16 + lane
            for t in range(d + ed + 1, aw):
                plsc.store_scatter(
                    vw, [p2v, rowi, jnp.full((16,), t, jnp.int32)], zf)
            return c
        lax.fori_loop(0, 2 * ngrp, zrow, 0)

        def issue_gathers(jj, p):
            slot = lax.rem(jj, 2)
            isl = lax.rem(jj, 3)
            pltpu.async_copy(v_hbm.at[idx6.at[isl]], vr.at[slot], smv)
            if p == 0:
                pltpu.async_copy(qx_hbm.at[idx6.at[3 + isl]], qxr.at[slot], smq)
                pltpu.async_copy(k_hbm.at[idx6.at[isl]], kr.at[slot], smk)
            pltpu.async_copy(ea_hbm.at[pl.ds(ebase + jj * _BLK, _BLK)],
                             vw.at[slot, :, pl.ds(d, ed)], smea)

        def drain_gathers(p):
            pltpu.make_async_copy(v_hbm.at[pl.ds(0, _BLK)],
                                  vr.at[0], smv).wait()
            if p == 0:
                pltpu.make_async_copy(qx_hbm.at[pl.ds(0, _BLK)],
                                      qxr.at[0], smq).wait()
                pltpu.make_async_copy(k_hbm.at[pl.ds(0, _BLK)],
                                      kr.at[0], smk).wait()
            pltpu.make_async_copy(ea_hbm.at[pl.ds(0, _BLK)],
                                  vw.at[0, :, pl.ds(d, ed)], smea).wait()

        def drain_scatter():
            pltpu.make_async_copy(zv_hbm.at[pl.ds(0, _BLK)],
                                  vw.at[0], smsc).wait()

        for p in range(2):
            lo = p * half
            pltpu.sync_copy(zv_hbm.at[pl.ds(sid * rpt, rpt)],
                            acc_sp.at[pl.ds(sid * rpt, rpt)])
            if rem:
                @pl.when(sid == _NS - 1)
                def _zero_tail():
                    pltpu.sync_copy(zv_hbm.at[pl.ds(rpt * _NS, rem)],
                                    acc_sp.at[pl.ds(rpt * _NS, rem)])
            plsc.subcore_barrier()
            pltpu.sync_copy(src_hbm.at[wid, pl.ds(0, 3)], idx6.at[pl.ds(0, 3)])
            pltpu.sync_copy(dst_hbm.at[wid, pl.ds(0, 3)], idx6.at[pl.ds(3, 3)])
            issue_gathers(0, p)

            def block(j, carry):
                par = lax.rem(j, 2)
                fpar = jnp.full((16,), par, jnp.int32)
                drain_gathers(p)

                @pl.when(j >= 1)
                def _drain_sc():
                    drain_scatter()

                @pl.when((j >= 2) & (j + 1 < nblk))
                def _drain_idx():
                    pltpu.make_async_copy(src_hbm.at[wid, 0],
                                          idx6.at[0], smsi).wait()
                    pltpu.make_async_copy(dst_hbm.at[wid, 0],
                                          idx6.at[0], smdi).wait()

                @pl.when(j + 1 < nblk)
                def _issue_next():
                    issue_gathers(j + 1, p)

                gid0 = ebase + j * _BLK

                def grp(g, carry2):
                    g16 = g * 16
                    rowi = g16 + lane
                    dstg = idx6[3 + lax.rem(j, 3), pl.ds(g16, 16)]
                    if p == 0:
                        @plsc.parallel_loop(0, 16, step=2, carry=jnp.int32(0))
                        def dotl(i0, c3):
                            for t in range(2):
                                i = i0 + t
                                r = g16 + i
                                a0 = (qxr[par, r, pl.ds(0, 16)]
                                      * kr[par, r, pl.ds(0, 16)])
                                a1 = (qxr[par, r, pl.ds(16, 16)]
                                      * kr[par, r, pl.ds(16, 16)])
                                for c in range(2, d // 16):
                                    pr = (qxr[par, r, pl.ds(c * 16, 16)]
                                          * kr[par, r, pl.ds(c * 16, 16)])
                                    if c % 2 == 0:
                                        a0 = a0 + pr
                                    else:
                                        a1 = a1 + pr
                                a0 = a0 + (qxr[par, r, pl.ds(d, 16)]
                                           * vw[par, r, pl.ds(d, 16)])
                                pbuf[i, pl.ds(0, 16)] = a0 + a1
                            return c3
                        sv0 = zf
                        sv1 = zf
                        for l in range(0, 16, 2):
                            sv0 = sv0 + plsc.load_gather(
                                pbuf, [lane, jnp.full((16,), l, jnp.int32)])
                            sv1 = sv1 + plsc.load_gather(
                                pbuf, [lane, jnp.full((16,), l + 1, jnp.int32)])
                        ids = gid0 + rowi
                        sv = jnp.where(ids < e,
                                       jnp.exp((sv0 + sv1) * inv), 0.0)
                        sbuf[j, pl.ds(g16, 16)] = sv
                    else:
                        sv = sbuf[j, pl.ds(g16, 16)]
                    inb = (dstg >= lo) & (dstg < lo + half)
                    svp = jnp.where(inb, sv, 0.0)
                    idx6[6, pl.ds(g16, 16)] = jnp.where(inb, dstg - lo, 0)

                    @plsc.parallel_loop(0, 16, step=2, carry=jnp.int32(0))
                    def scl(i0, c3):
                        for t in range(2):
                            i = i0 + t
                            r = g16 + i
                            bc = lax.gather(
                                svp, jnp.full((16, 1), i, jnp.int32),
                                _BCAST_DNUMS, (1,),
                                mode=lax.GatherScatterMode.PROMISE_IN_BOUNDS)
                            for c in range(d // 16):
                                vw[par, r, pl.ds(c * 16, 16)] = (
                                    vr[par, r, pl.ds(c * 16, 16)] * bc)
                            vw[par, r, pl.ds(d, 16)] = (
                                vw[par, r, pl.ds(d, 16)] * bc)
                        return c3
                    plsc.store_scatter(
                        vw, [fpar, rowi, jnp.full((16,), d + ed, jnp.int32)],
                        svp)
                    return carry2
                lax.fori_loop(0, ngrp, grp, 0)

                pltpu.async_copy(vw.at[par], acc_sp.at[idx6.at[6]], smsc,
                                 add=True)
                @pl.when(j + 3 < nblk)
                def _load_next():
                    nsl = lax.rem(j, 3)
                    pltpu.async_copy(src_hbm.at[wid, j + 3],
                                     idx6.at[nsl], smsi)
                    pltpu.async_copy(dst_hbm.at[wid, j + 3],
                                     idx6.at[3 + nsl], smdi)
                return carry
            lax.fori_loop(0, nblk, block, 0)
            drain_scatter()
            plsc.subcore_barrier()
            pltpu.sync_copy(acc_sp.at[pl.ds(sid * rpt, rpt)],
                            acc_out.at[cid, pl.ds(lo + sid * rpt, rpt)])
            if rem:
                @pl.when(sid == _NS - 1)
                def _spill_tail():
                    pltpu.sync_copy(
                        acc_sp.at[pl.ds(rpt * _NS, rem)],
                        acc_out.at[cid, pl.ds(lo + rpt * _NS, rem)])

    return edge_kernel


def kernel(x, pe, edge_index, edge_attr, batch, params):
    n, d = x.shape
    e = edge_index.shape[1]
    ed = edge_attr.shape[1]
    aw = d + ed + 8
    layers = params['layers']
    ec_pad = -(-e // (_NW * 2 * _BLK)) * 2 * _BLK
    pad = ec_pad * _NW - e

    src_r = jnp.pad(edge_index[0], (0, pad)).reshape(_NW, ec_pad // _BLK, _BLK)
    dst_r = jnp.pad(edge_index[1], (0, pad)).reshape(_NW, ec_pad // _BLK, _BLK)
    ea_pad = jnp.pad(edge_attr, ((0, pad), (0, 0)))
    zv = jnp.zeros((n, aw), jnp.float32)

    edge_fn = _make_edge_kernel(n, d, ed, e, ec_pad)

    p1, p2 = layers[0], layers[1]
    qx, k, v, sk = _tc_pre(x, p1)
    ws = {'We_comb': jnp.stack([p1['We'], p2['We']])}
    for name in ('Wq', 'bq', 'Wk', 'bk', 'Wv', 'bv', 'Wskip', 'bskip', 'We'):
        ws[name] = jnp.stack([p2[name], p2[name]])

    def step(carry, w):
        qx, k, v, sk, _ = carry
        acc = edge_fn(qx, k, v, ea_pad, src_r, dst_r, zv)
        h, qx2, k2, v2, sk2 = _tc_mid(acc, sk, w['We_comb'], w, aw)
        return (qx2, k2, v2, sk2, h), None

    carry, _ = lax.scan(step, (qx, k, v, sk, x), ws)
    return _tc_mlp(carry[4], params['mlp'])


# speedup vs baseline: 2.5762x; 1.0606x over previous
import functools
import math

import jax
import jax.numpy as jnp
from jax import lax
from jax.experimental import pallas as pl
from jax.experimental.pallas import tpu as pltpu
from jax.experimental.pallas import tpu_sc as plsc

_NC = 2
_NS = 16
_NW = _NC * _NS
_BLK = 64
_ROWB = 1000
_BCAST_DNUMS = lax.GatherDimensionNumbers(
    offset_dims=(), collapsed_slice_dims=(0,), start_index_map=(0,))


def _leaky(x):
    return jnp.where(x >= 0, x, 0.01 * x)


def _proj(h, wq, bq, wk, bk, wv, bv, wsk, bsk, we2, qx_o, k_o, v_o, sk_o, d):
    q = jnp.dot(h, wq[...], preferred_element_type=jnp.float32) + bq[...]
    qx_o[:, :d] = q
    qx_o[:, d:] = lax.dot_general(q, we2[...], (((1,), (1,)), ((), ())),
                                  preferred_element_type=jnp.float32)
    k_o[...] = jnp.dot(h, wk[...], preferred_element_type=jnp.float32) + bk[...]
    v_o[...] = jnp.dot(h, wv[...], preferred_element_type=jnp.float32) + bv[...]
    sk_o[...] = jnp.dot(h, wsk[...], preferred_element_type=jnp.float32) + bsk[...]


def _tc_pre_body(x_ref, wq, bq, wk, bk, wv, bv, wsk, bsk, we,
                 qx_o, k_o, v_o, sk_o):
    d = x_ref.shape[1]
    _proj(x_ref[...], wq, bq, wk, bk, wv, bv, wsk, bsk, we,
          qx_o, k_o, v_o, sk_o, d)


def _tc_pre(x, p):
    n, d = x.shape
    hc = p['Wq'].shape[1]
    ed = p['We'].shape[0]
    grid = (n // _ROWB,)
    full = lambda *s: pl.BlockSpec(s, lambda i: (0,) * len(s))
    rb = pl.BlockSpec((_ROWB, d), lambda i: (i, 0))
    out_rb = pl.BlockSpec((_ROWB, hc), lambda i: (i, 0))
    return pl.pallas_call(
        _tc_pre_body,
        grid=grid,
        in_specs=[rb, full(d, hc), full(1, hc), full(d, hc), full(1, hc),
                  full(d, hc), full(1, hc), full(d, hc), full(1, hc),
                  full(ed, hc)],
        out_specs=[pl.BlockSpec((_ROWB, hc + ed), lambda i: (i, 0)),
                   out_rb, out_rb, out_rb],
        out_shape=[jax.ShapeDtypeStruct((n, hc + ed), jnp.float32)]
        + [jax.ShapeDtypeStruct((n, hc), jnp.float32)] * 3,
    )(x, p['Wq'], p['bq'].reshape(1, -1), p['Wk'], p['bk'].reshape(1, -1),
      p['Wv'], p['bv'].reshape(1, -1), p['Wskip'], p['bskip'].reshape(1, -1),
      p['We'])


def _combine(acc_ref, sk_ref, we_ref):
    d = sk_ref.shape[1]
    ed = we_ref.shape[0]
    a = acc_ref[0] + acc_ref[1]
    den = a[:, d + ed:d + ed + 1] + 1e-16
    h = (a[:, :d] + jnp.dot(a[:, d:d + ed], we_ref[...],
                            preferred_element_type=jnp.float32)) / den
    return _leaky(h + sk_ref[...])


def _tc_mid_body(acc_ref, sk_ref, we_ref,
                 wq, bq, wk, bk, wv, bv, wsk, bsk, we2,
                 h_o, qx_o, k_o, v_o, sk_o):
    h = _combine(acc_ref, sk_ref, we_ref)
    h_o[...] = h
    _proj(h, wq, bq, wk, bk, wv, bv, wsk, bsk, we2,
          qx_o, k_o, v_o, sk_o, sk_ref.shape[1])


def _tc_mid(acc, sk, we_prev, p, aw):
    n = sk.shape[0]
    d = sk.shape[1]
    hc = p['Wq'].shape[1]
    ed = we_prev.shape[0]
    grid = (n // _ROWB,)
    full = lambda *s: pl.BlockSpec(s, lambda i: (0,) * len(s))
    rb = pl.BlockSpec((_ROWB, d), lambda i: (i, 0))
    out_rb = pl.BlockSpec((_ROWB, hc), lambda i: (i, 0))
    return pl.pallas_call(
        _tc_mid_body,
        grid=grid,
        in_specs=[pl.BlockSpec((_NC, _ROWB, aw), lambda i: (0, i, 0)),
                  rb, full(ed, d),
                  full(d, hc), full(1, hc), full(d, hc), full(1, hc),
                  full(d, hc), full(1, hc), full(d, hc), full(1, hc),
                  full(ed, hc)],
        out_specs=[rb, pl.BlockSpec((_ROWB, hc + ed), lambda i: (i, 0)),
                   out_rb, out_rb, out_rb],
        out_shape=[jax.ShapeDtypeStruct((n, d), jnp.float32),
                   jax.ShapeDtypeStruct((n, hc + ed), jnp.float32)]
        + [jax.ShapeDtypeStruct((n, hc), jnp.float32)] * 3,
    )(acc, sk, we_prev, p['Wq'], p['bq'].reshape(1, -1),
      p['Wk'], p['bk'].reshape(1, -1), p['Wv'], p['bv'].reshape(1, -1),
      p['Wskip'], p['bskip'].reshape(1, -1), p['We'])


def _tc_mlp_body(h_ref, w1, b1, w2, b2, y_o):
    h = _leaky(jnp.dot(h_ref[...], w1[...],
                       preferred_element_type=jnp.float32) + b1[...])
    y_o[...] = jnp.dot(h, w2[...], preferred_element_type=jnp.float32) + b2[...]


def _tc_mlp(h, mlp):
    n, d = h.shape
    hid = mlp['W1'].shape[1]
    out = mlp['W2'].shape[1]
    grid = (n // _ROWB,)
    full = lambda *s: pl.BlockSpec(s, lambda i: (0,) * len(s))
    return pl.pallas_call(
        _tc_mlp_body,
        grid=grid,
        in_specs=[pl.BlockSpec((_ROWB, d), lambda i: (i, 0)),
                  full(d, hid), full(1, hid), full(hid, out), full(1, out)],
        out_specs=pl.BlockSpec((_ROWB, out), lambda i: (i, 0)),
        out_shape=jax.ShapeDtypeStruct((n, out), jnp.float32),
    )(h, mlp['W1'], mlp['b1'].reshape(1, -1),
      mlp['W2'], mlp['b2'].reshape(1, -1))


@functools.cache
def _make_edge_kernel(n, d, ed, e, ec_pad):
    nblk = ec_pad // _BLK
    half = n // 2
    qw = d + ed
    aw = d + ed + 8
    rpt = (half // _NS) // 8 * 8
    rem = half - rpt * _NS
    mesh = plsc.VectorSubcoreMesh(core_axis_name="c", subcore_axis_name="s",
                                  num_cores=_NC, num_subcores=_NS)
    inv = 1.0 / math.sqrt(d)

    @functools.partial(
        pl.kernel,
        out_type=jax.ShapeDtypeStruct((_NC, n, aw), jnp.float32),
        mesh=mesh,
        compiler_params=pltpu.CompilerParams(needs_layout_passes=False,
                                             use_tc_tiling_on_sc=False),
        scratch_types=[
            pltpu.VMEM((8, _BLK), jnp.int32),
            pltpu.VMEM((2, _BLK, qw), jnp.float32),
            pltpu.VMEM((2, _BLK, d), jnp.float32),
            pltpu.VMEM((2, _BLK, d), jnp.float32),
            pltpu.VMEM((2, _BLK, aw), jnp.float32),
            pltpu.VMEM((nblk, _BLK), jnp.float32),
            pltpu.VMEM((16, 17), jnp.float32),
            pltpu.VMEM_SHARED((half, aw), jnp.float32),
            pltpu.SemaphoreType.DMA,
            pltpu.SemaphoreType.DMA,
            pltpu.SemaphoreType.DMA,
            pltpu.SemaphoreType.DMA,
            pltpu.SemaphoreType.DMA,
            pltpu.SemaphoreType.DMA,
            pltpu.SemaphoreType.DMA,
        ],
    )
    def edge_kernel(qx_hbm, k_hbm, v_hbm, ea_hbm, src_hbm, dst_hbm, zv_hbm,
                    acc_out,
                    idx6, qxr, kr, vr, vw, sbuf, pbuf,
                    acc_sp, smq, smk, smv, smea, smsc, smsi, smdi):
        cid = lax.axis_index("c")
        sid = lax.axis_index("s")
        wid = cid * _NS + sid
        ebase = wid * ec_pad
        lane = lax.iota(jnp.int32, 16)
        zf = jnp.zeros((16,), jnp.float32)
        ngrp = _BLK // 16

        def zrow(i, c):
            p2v = jnp.full((16,), lax.div(i, ngrp), jnp.int32)
            rowi = lax.rem(i, ngrp) * 16 + lane
            for t in range(d + ed + 1, aw):
                plsc.store_scatter(
                    vw, [p2v, rowi, jnp.full((16,), t, jnp.int32)], zf)
            return c
        lax.fori_loop(0, 2 * ngrp, zrow, 0)

        def issue_gathers(jj, p):
            slot = lax.rem(jj, 2)
            isl = lax.rem(jj, 3)
            pltpu.async_copy(v_hbm.at[idx6.at[isl]], vr.at[slot], smv)
            if p == 0:
                pltpu.async_copy(qx_hbm.at[idx6.at[3 + isl]], qxr.at[slot], smq)
                pltpu.async_copy(k_hbm.at[idx6.at[isl]], kr.at[slot], smk)
            pltpu.async_copy(ea_hbm.at[pl.ds(ebase + jj * _BLK, _BLK)],
                             vw.at[slot, :, pl.ds(d, ed)], smea)

        def drain_gathers(p):
            pltpu.make_async_copy(v_hbm.at[pl.ds(0, _BLK)],
                                  vr.at[0], smv).wait()
            if p == 0:
                pltpu.make_async_copy(qx_hbm.at[pl.ds(0, _BLK)],
                                      qxr.at[0], smq).wait()
                pltpu.make_async_copy(k_hbm.at[pl.ds(0, _BLK)],
                                      kr.at[0], smk).wait()
            pltpu.make_async_copy(ea_hbm.at[pl.ds(0, _BLK)],
                                  vw.at[0, :, pl.ds(d, ed)], smea).wait()

        def drain_scatter():
            pltpu.make_async_copy(zv_hbm.at[pl.ds(0, _BLK)],
                                  vw.at[0], smsc).wait()

        for p in range(2):
            lo = p * half
            pltpu.sync_copy(zv_hbm.at[pl.ds(sid * rpt, rpt)],
                            acc_sp.at[pl.ds(sid * rpt, rpt)])
            if rem:
                @pl.when(sid == _NS - 1)
                def _zero_tail():
                    pltpu.sync_copy(zv_hbm.at[pl.ds(rpt * _NS, rem)],
                                    acc_sp.at[pl.ds(rpt * _NS, rem)])
            plsc.subcore_barrier()
            pltpu.sync_copy(src_hbm.at[wid, pl.ds(0, 3)], idx6.at[pl.ds(0, 3)])
            pltpu.sync_copy(dst_hbm.at[wid, pl.ds(0, 3)], idx6.at[pl.ds(3, 3)])
            issue_gathers(0, p)

            def block(j, carry):
                par = lax.rem(j, 2)
                fpar = jnp.full((16,), par, jnp.int32)
                drain_gathers(p)

                @pl.when(j >= 2)
                def _drain_sc():
                    drain_scatter()

                @pl.when((j >= 2) & (j + 1 < nblk))
                def _drain_idx():
                    pltpu.make_async_copy(src_hbm.at[wid, 0],
                                          idx6.at[0], smsi).wait()
                    pltpu.make_async_copy(dst_hbm.at[wid, 0],
                                          idx6.at[0], smdi).wait()

                @pl.when(j + 1 < nblk)
                def _issue_next():
                    issue_gathers(j + 1, p)

                gid0 = ebase + j * _BLK

                def grp(g, carry2):
                    g16 = g * 16
                    rowi = g16 + lane
                    dstg = idx6[3 + lax.rem(j, 3), pl.ds(g16, 16)]
                    if p == 0:
                        @plsc.parallel_loop(0, 16, step=2, carry=jnp.int32(0))
                        def dotl(i0, c3):
                            for t in range(2):
                                i = i0 + t
                                r = g16 + i
                                a0 = (qxr[par, r, pl.ds(0, 16)]
                                      * kr[par, r, pl.ds(0, 16)])
                                a1 = (qxr[par, r, pl.ds(16, 16)]
                                      * kr[par, r, pl.ds(16, 16)])
                                for c in range(2, d // 16):
                                    pr = (qxr[par, r, pl.ds(c * 16, 16)]
                                          * kr[par, r, pl.ds(c * 16, 16)])
                                    if c % 2 == 0:
                                        a0 = a0 + pr
                                    else:
                                        a1 = a1 + pr
                                a0 = a0 + (qxr[par, r, pl.ds(d, 16)]
                                           * vw[par, r, pl.ds(d, 16)])
                                pbuf[i, pl.ds(0, 16)] = a0 + a1
                            return c3
                        sv0 = zf
                        sv1 = zf
                        for l in range(0, 16, 2):
                            sv0 = sv0 + plsc.load_gather(
                                pbuf, [lane, jnp.full((16,), l, jnp.int32)])
                            sv1 = sv1 + plsc.load_gather(
                                pbuf, [lane, jnp.full((16,), l + 1, jnp.int32)])
                        ids = gid0 + rowi
                        sv = jnp.where(ids < e,
                                       jnp.exp((sv0 + sv1) * inv), 0.0)
                        sbuf[j, pl.ds(g16, 16)] = sv
                    else:
                        sv = sbuf[j, pl.ds(g16, 16)]
                    inb = (dstg >= lo) & (dstg < lo + half)
                    svp = jnp.where(inb, sv, 0.0)
                    idx6[6 + par, pl.ds(g16, 16)] = jnp.where(inb, dstg - lo, 0)

                    @plsc.parallel_loop(0, 16, step=2, carry=jnp.int32(0))
                    def scl(i0, c3):
                        for t in range(2):
                            i = i0 + t
                            r = g16 + i
                            bc = lax.gather(
                                svp, jnp.full((16, 1), i, jnp.int32),
                                _BCAST_DNUMS, (1,),
                                mode=lax.GatherScatterMode.PROMISE_IN_BOUNDS)
                            for c in range(d // 16):
                                vw[par, r, pl.ds(c * 16, 16)] = (
                                    vr[par, r, pl.ds(c * 16, 16)] * bc)
                            vw[par, r, pl.ds(d, 16)] = (
                                vw[par, r, pl.ds(d, 16)] * bc)
                        return c3
                    plsc.store_scatter(
                        vw, [fpar, rowi, jnp.full((16,), d + ed, jnp.int32)],
                        svp)
                    return carry2
                lax.fori_loop(0, ngrp, grp, 0)

                pltpu.async_copy(vw.at[par], acc_sp.at[idx6.at[6 + par]], smsc,
                                 add=True)
                @pl.when(j + 3 < nblk)
                def _load_next():
                    nsl = lax.rem(j, 3)
                    pltpu.async_copy(src_hbm.at[wid, j + 3],
                                     idx6.at[nsl], smsi)
                    pltpu.async_copy(dst_hbm.at[wid, j + 3],
                                     idx6.at[3 + nsl], smdi)
                return carry
            lax.fori_loop(0, nblk, block, 0)
            drain_scatter()
            drain_scatter()
            plsc.subcore_barrier()
            pltpu.sync_copy(acc_sp.at[pl.ds(sid * rpt, rpt)],
                            acc_out.at[cid, pl.ds(lo + sid * rpt, rpt)])
            if rem:
                @pl.when(sid == _NS - 1)
                def _spill_tail():
                    pltpu.sync_copy(
                        acc_sp.at[pl.ds(rpt * _NS, rem)],
                        acc_out.at[cid, pl.ds(lo + rpt * _NS, rem)])

    return edge_kernel


def kernel(x, pe, edge_index, edge_attr, batch, params):
    n, d = x.shape
    e = edge_index.shape[1]
    ed = edge_attr.shape[1]
    aw = d + ed + 8
    layers = params['layers']
    ec_pad = -(-e // (_NW * 2 * _BLK)) * 2 * _BLK
    pad = ec_pad * _NW - e

    src_r = jnp.pad(edge_index[0], (0, pad)).reshape(_NW, ec_pad // _BLK, _BLK)
    dst_r = jnp.pad(edge_index[1], (0, pad)).reshape(_NW, ec_pad // _BLK, _BLK)
    ea_pad = jnp.pad(edge_attr, ((0, pad), (0, 0)))
    zv = jnp.zeros((n, aw), jnp.float32)

    edge_fn = _make_edge_kernel(n, d, ed, e, ec_pad)

    p1, p2 = layers[0], layers[1]
    qx, k, v, sk = _tc_pre(x, p1)
    ws = {'We_comb': jnp.stack([p1['We'], p2['We']])}
    for name in ('Wq', 'bq', 'Wk', 'bk', 'Wv', 'bv', 'Wskip', 'bskip', 'We'):
        ws[name] = jnp.stack([p2[name], p2[name]])

    def step(carry, w):
        qx, k, v, sk, _ = carry
        acc = edge_fn(qx, k, v, ea_pad, src_r, dst_r, zv)
        h, qx2, k2, v2, sk2 = _tc_mid(acc, sk, w['We_comb'], w, aw)
        return (qx2, k2, v2, sk2, h), None

    carry, _ = lax.scan(step, (qx, k, v, sk, x), ws)
    return _tc_mlp(carry[4], params['mlp'])


# speedup vs baseline: 2.8135x; 1.0921x over previous
import functools
import math

import jax
import jax.numpy as jnp
from jax import lax
from jax.experimental import pallas as pl
from jax.experimental.pallas import tpu as pltpu
from jax.experimental.pallas import tpu_sc as plsc

_NC = 2
_NS = 16
_NW = _NC * _NS
_BLK = 64
_ROWB = 1000
_BCAST_DNUMS = lax.GatherDimensionNumbers(
    offset_dims=(), collapsed_slice_dims=(0,), start_index_map=(0,))


def _leaky(x):
    return jnp.where(x >= 0, x, 0.01 * x)


def _proj(h, wq, bq, wk, bk, wv, bv, wsk, bsk, we2,
          qx_o, qe_o, k_o, v_o, sk_o, d):
    q = jnp.dot(h, wq[...], preferred_element_type=jnp.float32) + bq[...]
    qx_o[...] = q.astype(jnp.bfloat16)
    qe_o[...] = lax.dot_general(q, we2[...], (((1,), (1,)), ((), ())),
                                preferred_element_type=jnp.float32)
    k_o[...] = (jnp.dot(h, wk[...], preferred_element_type=jnp.float32)
                + bk[...]).astype(jnp.bfloat16)
    v_o[...] = jnp.dot(h, wv[...], preferred_element_type=jnp.float32) + bv[...]
    sk_o[...] = jnp.dot(h, wsk[...], preferred_element_type=jnp.float32) + bsk[...]


def _tc_pre_body(x_ref, wq, bq, wk, bk, wv, bv, wsk, bsk, we,
                 qx_o, qe_o, k_o, v_o, sk_o):
    d = x_ref.shape[1]
    _proj(x_ref[...], wq, bq, wk, bk, wv, bv, wsk, bsk, we,
          qx_o, qe_o, k_o, v_o, sk_o, d)


def _tc_pre(x, p):
    n, d = x.shape
    hc = p['Wq'].shape[1]
    ed = p['We'].shape[0]
    grid = (n // _ROWB,)
    full = lambda *s: pl.BlockSpec(s, lambda i: (0,) * len(s))
    rb = pl.BlockSpec((_ROWB, d), lambda i: (i, 0))
    out_rb = pl.BlockSpec((_ROWB, hc), lambda i: (i, 0))
    return pl.pallas_call(
        _tc_pre_body,
        grid=grid,
        in_specs=[rb, full(d, hc), full(1, hc), full(d, hc), full(1, hc),
                  full(d, hc), full(1, hc), full(d, hc), full(1, hc),
                  full(ed, hc)],
        out_specs=[out_rb, pl.BlockSpec((_ROWB, ed), lambda i: (i, 0)),
                   out_rb, out_rb, out_rb],
        out_shape=[jax.ShapeDtypeStruct((n, hc), jnp.bfloat16),
                   jax.ShapeDtypeStruct((n, ed), jnp.float32),
                   jax.ShapeDtypeStruct((n, hc), jnp.bfloat16)]
        + [jax.ShapeDtypeStruct((n, hc), jnp.float32)] * 2,
    )(x, p['Wq'], p['bq'].reshape(1, -1), p['Wk'], p['bk'].reshape(1, -1),
      p['Wv'], p['bv'].reshape(1, -1), p['Wskip'], p['bskip'].reshape(1, -1),
      p['We'])


def _combine(acc_ref, sk_ref, we_ref):
    d = sk_ref.shape[1]
    ed = we_ref.shape[0]
    a = acc_ref[0] + acc_ref[1]
    den = a[:, d + ed:d + ed + 1] + 1e-16
    h = (a[:, :d] + jnp.dot(a[:, d:d + ed], we_ref[...],
                            preferred_element_type=jnp.float32)) / den
    return _leaky(h + sk_ref[...])


def _tc_mid_body(acc_ref, sk_ref, we_ref,
                 wq, bq, wk, bk, wv, bv, wsk, bsk, we2,
                 h_o, qx_o, qe_o, k_o, v_o, sk_o):
    h = _combine(acc_ref, sk_ref, we_ref)
    h_o[...] = h
    _proj(h, wq, bq, wk, bk, wv, bv, wsk, bsk, we2,
          qx_o, qe_o, k_o, v_o, sk_o, sk_ref.shape[1])


def _tc_mid(acc, sk, we_prev, p, aw):
    n = sk.shape[0]
    d = sk.shape[1]
    hc = p['Wq'].shape[1]
    ed = we_prev.shape[0]
    grid = (n // _ROWB,)
    full = lambda *s: pl.BlockSpec(s, lambda i: (0,) * len(s))
    rb = pl.BlockSpec((_ROWB, d), lambda i: (i, 0))
    out_rb = pl.BlockSpec((_ROWB, hc), lambda i: (i, 0))
    return pl.pallas_call(
        _tc_mid_body,
        grid=grid,
        in_specs=[pl.BlockSpec((_NC, _ROWB, aw), lambda i: (0, i, 0)),
                  rb, full(ed, d),
                  full(d, hc), full(1, hc), full(d, hc), full(1, hc),
                  full(d, hc), full(1, hc), full(d, hc), full(1, hc),
                  full(ed, hc)],
        out_specs=[rb, out_rb, pl.BlockSpec((_ROWB, ed), lambda i: (i, 0)),
                   out_rb, out_rb, out_rb],
        out_shape=[jax.ShapeDtypeStruct((n, d), jnp.float32),
                   jax.ShapeDtypeStruct((n, hc), jnp.bfloat16),
                   jax.ShapeDtypeStruct((n, ed), jnp.float32),
                   jax.ShapeDtypeStruct((n, hc), jnp.bfloat16)]
        + [jax.ShapeDtypeStruct((n, hc), jnp.float32)] * 2,
    )(acc, sk, we_prev, p['Wq'], p['bq'].reshape(1, -1),
      p['Wk'], p['bk'].reshape(1, -1), p['Wv'], p['bv'].reshape(1, -1),
      p['Wskip'], p['bskip'].reshape(1, -1), p['We'])


def _tc_mlp_body(h_ref, w1, b1, w2, b2, y_o):
    h = _leaky(jnp.dot(h_ref[...], w1[...],
                       preferred_element_type=jnp.float32) + b1[...])
    y_o[...] = jnp.dot(h, w2[...], preferred_element_type=jnp.float32) + b2[...]


def _tc_mlp(h, mlp):
    n, d = h.shape
    hid = mlp['W1'].shape[1]
    out = mlp['W2'].shape[1]
    grid = (n // _ROWB,)
    full = lambda *s: pl.BlockSpec(s, lambda i: (0,) * len(s))
    return pl.pallas_call(
        _tc_mlp_body,
        grid=grid,
        in_specs=[pl.BlockSpec((_ROWB, d), lambda i: (i, 0)),
                  full(d, hid), full(1, hid), full(hid, out), full(1, out)],
        out_specs=pl.BlockSpec((_ROWB, out), lambda i: (i, 0)),
        out_shape=jax.ShapeDtypeStruct((n, out), jnp.float32),
    )(h, mlp['W1'], mlp['b1'].reshape(1, -1),
      mlp['W2'], mlp['b2'].reshape(1, -1))


@functools.cache
def _make_edge_kernel(n, d, ed, e, ec_pad):
    nblk = ec_pad // _BLK
    half = n // 2
    aw = d + ed + 8
    rpt = (half // _NS) // 8 * 8
    rem = half - rpt * _NS
    mesh = plsc.VectorSubcoreMesh(core_axis_name="c", subcore_axis_name="s",
                                  num_cores=_NC, num_subcores=_NS)
    inv = 1.0 / math.sqrt(d)

    @functools.partial(
        pl.kernel,
        out_type=jax.ShapeDtypeStruct((_NC, n, aw), jnp.float32),
        mesh=mesh,
        compiler_params=pltpu.CompilerParams(needs_layout_passes=False,
                                             use_tc_tiling_on_sc=False),
        scratch_types=[
            pltpu.VMEM((8, _BLK), jnp.int32),
            pltpu.VMEM((2, _BLK, d), jnp.bfloat16),
            pltpu.VMEM((2, _BLK, ed), jnp.float32),
            pltpu.VMEM((2, _BLK, d), jnp.bfloat16),
            pltpu.VMEM((2, _BLK, d), jnp.float32),
            pltpu.VMEM((2, _BLK, aw), jnp.float32),
            pltpu.VMEM((nblk, _BLK), jnp.float32),
            pltpu.VMEM((16, 17), jnp.float32),
            pltpu.VMEM_SHARED((half, aw), jnp.float32),
            pltpu.SemaphoreType.DMA,
            pltpu.SemaphoreType.DMA,
            pltpu.SemaphoreType.DMA,
            pltpu.SemaphoreType.DMA,
            pltpu.SemaphoreType.DMA,
            pltpu.SemaphoreType.DMA,
            pltpu.SemaphoreType.DMA,
        ],
    )
    def edge_kernel(qx_hbm, qe_hbm, k_hbm, v_hbm, ea_hbm, src_hbm, dst_hbm,
                    zv_hbm, acc_out,
                    idx6, qxr, qer, kr, vr, vw, sbuf, pbuf,
                    acc_sp, smq, smk, smv, smea, smsc, smsi, smdi):
        cid = lax.axis_index("c")
        sid = lax.axis_index("s")
        wid = cid * _NS + sid
        ebase = wid * ec_pad
        lane = lax.iota(jnp.int32, 16)
        zf = jnp.zeros((16,), jnp.float32)
        ngrp = _BLK // 16

        def zrow(i, c):
            p2v = jnp.full((16,), lax.div(i, ngrp), jnp.int32)
            rowi = lax.rem(i, ngrp) * 16 + lane
            for t in range(d + ed + 1, aw):
                plsc.store_scatter(
                    vw, [p2v, rowi, jnp.full((16,), t, jnp.int32)], zf)
            return c
        lax.fori_loop(0, 2 * ngrp, zrow, 0)

        def issue_gathers(jj, p):
            slot = lax.rem(jj, 2)
            isl = lax.rem(jj, 3)
            pltpu.async_copy(v_hbm.at[idx6.at[isl]], vr.at[slot], smv)
            if p == 0:
                pltpu.async_copy(qx_hbm.at[idx6.at[3 + isl]], qxr.at[slot], smq)
                pltpu.async_copy(qe_hbm.at[idx6.at[3 + isl]], qer.at[slot], smq)
                pltpu.async_copy(k_hbm.at[idx6.at[isl]], kr.at[slot], smk)
            pltpu.async_copy(ea_hbm.at[pl.ds(ebase + jj * _BLK, _BLK)],
                             vw.at[slot, :, pl.ds(d, ed)], smea)

        def drain_gathers(p):
            pltpu.make_async_copy(v_hbm.at[pl.ds(0, _BLK)],
                                  vr.at[0], smv).wait()
            if p == 0:
                pltpu.make_async_copy(qx_hbm.at[pl.ds(0, _BLK)],
                                      qxr.at[0], smq).wait()
                pltpu.make_async_copy(qe_hbm.at[pl.ds(0, _BLK)],
                                      qer.at[0], smq).wait()
                pltpu.make_async_copy(k_hbm.at[pl.ds(0, _BLK)],
                                      kr.at[0], smk).wait()
            pltpu.make_async_copy(ea_hbm.at[pl.ds(0, _BLK)],
                                  vw.at[0, :, pl.ds(d, ed)], smea).wait()

        def drain_scatter():
            pltpu.make_async_copy(zv_hbm.at[pl.ds(0, _BLK)],
                                  vw.at[0], smsc).wait()

        for p in range(2):
            lo = p * half
            pltpu.sync_copy(zv_hbm.at[pl.ds(sid * rpt, rpt)],
                            acc_sp.at[pl.ds(sid * rpt, rpt)])
            if rem:
                @pl.when(sid == _NS - 1)
                def _zero_tail():
                    pltpu.sync_copy(zv_hbm.at[pl.ds(rpt * _NS, rem)],
                                    acc_sp.at[pl.ds(rpt * _NS, rem)])
            plsc.subcore_barrier()
            pltpu.sync_copy(src_hbm.at[wid, pl.ds(0, 3)], idx6.at[pl.ds(0, 3)])
            pltpu.sync_copy(dst_hbm.at[wid, pl.ds(0, 3)], idx6.at[pl.ds(3, 3)])
            issue_gathers(0, p)

            def block(j, carry):
                par = lax.rem(j, 2)
                fpar = jnp.full((16,), par, jnp.int32)
                drain_gathers(p)

                @pl.when(j >= 2)
                def _drain_sc():
                    drain_scatter()

                @pl.when((j >= 2) & (j + 1 < nblk))
                def _drain_idx():
                    pltpu.make_async_copy(src_hbm.at[wid, 0],
                                          idx6.at[0], smsi).wait()
                    pltpu.make_async_copy(dst_hbm.at[wid, 0],
                                          idx6.at[0], smdi).wait()

                @pl.when(j + 1 < nblk)
                def _issue_next():
                    issue_gathers(j + 1, p)

                gid0 = ebase + j * _BLK

                def grp(g, carry2):
                    g16 = g * 16
                    rowi = g16 + lane
                    dstg = idx6[3 + lax.rem(j, 3), pl.ds(g16, 16)]
                    if p == 0:
                        @plsc.parallel_loop(0, 16, step=2, carry=jnp.int32(0))
                        def dotl(i0, c3):
                            for t in range(2):
                                i = i0 + t
                                r = g16 + i
                                a0 = zf
                                a1 = zf
                                for c in range(d // 32):
                                    pr = (qxr[par, r, pl.ds(c * 32, 32)]
                                          * kr[par, r, pl.ds(c * 32, 32)])
                                    e0, e1 = plsc.unpack(
                                        pr, format=plsc.PackFormat.INTERLEAVED)
                                    a0 = a0 + e0
                                    a1 = a1 + e1
                                a0 = a0 + (qer[par, r, :]
                                           * vw[par, r, pl.ds(d, 16)])
                                pbuf[i, pl.ds(0, 16)] = a0 + a1
                            return c3
                        sv0 = zf
                        sv1 = zf
                        for l in range(0, 16, 2):
                            sv0 = sv0 + plsc.load_gather(
                                pbuf, [lane, jnp.full((16,), l, jnp.int32)])
                            sv1 = sv1 + plsc.load_gather(
                                pbuf, [lane, jnp.full((16,), l + 1, jnp.int32)])
                        ids = gid0 + rowi
                        sv = jnp.where(ids < e,
                                       jnp.exp((sv0 + sv1) * inv), 0.0)
                        sbuf[j, pl.ds(g16, 16)] = sv
                    else:
                        sv = sbuf[j, pl.ds(g16, 16)]
                    inb = (dstg >= lo) & (dstg < lo + half)
                    svp = jnp.where(inb, sv, 0.0)
                    idx6[6 + par, pl.ds(g16, 16)] = jnp.where(inb, dstg - lo, 0)

                    @plsc.parallel_loop(0, 16, step=2, carry=jnp.int32(0))
                    def scl(i0, c3):
                        for t in range(2):
                            i = i0 + t
                            r = g16 + i
                            bc = lax.gather(
                                svp, jnp.full((16, 1), i, jnp.int32),
                                _BCAST_DNUMS, (1,),
                                mode=lax.GatherScatterMode.PROMISE_IN_BOUNDS)
                            for c in range(d // 16):
                                vw[par, r, pl.ds(c * 16, 16)] = (
                                    vr[par, r, pl.ds(c * 16, 16)] * bc)
                            vw[par, r, pl.ds(d, 16)] = (
                                vw[par, r, pl.ds(d, 16)] * bc)
                        return c3
                    plsc.store_scatter(
                        vw, [fpar, rowi, jnp.full((16,), d + ed, jnp.int32)],
                        svp)
                    return carry2
                lax.fori_loop(0, ngrp, grp, 0)

                pltpu.async_copy(vw.at[par], acc_sp.at[idx6.at[6 + par]], smsc,
                                 add=True)
                @pl.when(j + 3 < nblk)
                def _load_next():
                    nsl = lax.rem(j, 3)
                    pltpu.async_copy(src_hbm.at[wid, j + 3],
                                     idx6.at[nsl], smsi)
                    pltpu.async_copy(dst_hbm.at[wid, j + 3],
                                     idx6.at[3 + nsl], smdi)
                return carry
            lax.fori_loop(0, nblk, block, 0)
            drain_scatter()
            drain_scatter()
            plsc.subcore_barrier()
            pltpu.sync_copy(acc_sp.at[pl.ds(sid * rpt, rpt)],
                            acc_out.at[cid, pl.ds(lo + sid * rpt, rpt)])
            if rem:
                @pl.when(sid == _NS - 1)
                def _spill_tail():
                    pltpu.sync_copy(
                        acc_sp.at[pl.ds(rpt * _NS, rem)],
                        acc_out.at[cid, pl.ds(lo + rpt * _NS, rem)])

    return edge_kernel


def kernel(x, pe, edge_index, edge_attr, batch, params):
    n, d = x.shape
    e = edge_index.shape[1]
    ed = edge_attr.shape[1]
    aw = d + ed + 8
    layers = params['layers']
    ec_pad = -(-e // (_NW * 2 * _BLK)) * 2 * _BLK
    pad = ec_pad * _NW - e

    src_r = jnp.pad(edge_index[0], (0, pad)).reshape(_NW, ec_pad // _BLK, _BLK)
    dst_r = jnp.pad(edge_index[1], (0, pad)).reshape(_NW, ec_pad // _BLK, _BLK)
    ea_pad = jnp.pad(edge_attr, ((0, pad), (0, 0)))
    zv = jnp.zeros((n, aw), jnp.float32)

    edge_fn = _make_edge_kernel(n, d, ed, e, ec_pad)

    p1, p2 = layers[0], layers[1]
    qx, qe, k, v, sk = _tc_pre(x, p1)
    ws = {'We_comb': jnp.stack([p1['We'], p2['We']])}
    for name in ('Wq', 'bq', 'Wk', 'bk', 'Wv', 'bv', 'Wskip', 'bskip', 'We'):
        ws[name] = jnp.stack([p2[name], p2[name]])

    def step(carry, w):
        qx, qe, k, v, sk, _ = carry
        acc = edge_fn(qx, qe, k, v, ea_pad, src_r, dst_r, zv)
        h, qx2, qe2, k2, v2, sk2 = _tc_mid(acc, sk, w['We_comb'], w, aw)
        return (qx2, qe2, k2, v2, sk2, h), None

    carry, _ = lax.scan(step, (qx, qe, k, v, sk, x), ws)
    return _tc_mlp(carry[5], params['mlp'])
